# Initial kernel scaffold; baseline (speedup 1.0000x reference)
import jax
import jax.numpy as jnp
from jax.experimental import pallas as pl


def kernel(tokens, ln_w, ln_b, w, b):
    raise NotImplementedError("write your pallas kernel here")


# speedup vs baseline: 2.2606x; 2.2606x over previous
import functools

import jax
import jax.numpy as jnp
import numpy as np
from jax import lax
from jax.experimental import pallas as pl
from jax.experimental.pallas import tpu as pltpu

_DIM = 768
_KEEP = 1024
_ROWS_PER_BLK = 2048
_INV_DIM = np.float32(1.0 / 768.0)
_EPS = np.float32(1e-5)


def _score_body(x_ref, mu_ref, vs_ref, ln_w_ref, ln_b_ref, wpad_ref, b_ref,
                score_ref):
    x = x_ref[0]
    mu = mu_ref[0, 0][:, None]
    sigma = jnp.sqrt(vs_ref[0, 0] * _INV_DIM + _EPS)[:, None]
    h = (x - mu) / sigma * ln_w_ref[...] + ln_b_ref[...]
    s_full = jnp.dot(h, wpad_ref[...], preferred_element_type=jnp.float32)
    score_ref[0, 0] = s_full[:, 0] + b_ref[0]


def _scores(tokens, sumt, varsum, ln_w, ln_b, w, b):
    bsz, n, d = tokens.shape
    rows = bsz * n
    nblk = rows // _ROWS_PER_BLK
    x3 = tokens.reshape(nblk, _ROWS_PER_BLK, d)
    mu3 = (sumt * _INV_DIM).reshape(nblk, 1, _ROWS_PER_BLK)
    vs3 = varsum.reshape(nblk, 1, _ROWS_PER_BLK)
    wpad = jnp.zeros((d, 128), jnp.float32).at[:, 0].set(w[0])
    out = pl.pallas_call(
        _score_body,
        grid=(nblk,),
        in_specs=[
            pl.BlockSpec((1, _ROWS_PER_BLK, d), lambda i: (i, 0, 0)),
            pl.BlockSpec((1, 1, _ROWS_PER_BLK), lambda i: (i, 0, 0)),
            pl.BlockSpec((1, 1, _ROWS_PER_BLK), lambda i: (i, 0, 0)),
            pl.BlockSpec((d,), lambda i: (0,)),
            pl.BlockSpec((d,), lambda i: (0,)),
            pl.BlockSpec((d, 128), lambda i: (0, 0)),
            pl.BlockSpec((1,), lambda i: (0,)),
        ],
        out_specs=pl.BlockSpec((1, 1, _ROWS_PER_BLK), lambda i: (i, 0, 0)),
        out_shape=jax.ShapeDtypeStruct((nblk, 1, _ROWS_PER_BLK), jnp.float32),
    )(x3, mu3, vs3, ln_w, ln_b, wpad, b)
    return out.reshape(bsz, n)


def _topk_body(score_ref, idx_ref):
    s = score_ref[...]
    bsz, n = s.shape
    bits = jax.lax.bitcast_convert_type(s, jnp.int32)
    key = jnp.where(bits < 0, jnp.int32(0x7FFFFFFF) ^ bits, bits)
    idx = jax.lax.broadcasted_iota(jnp.int32, (bsz, n), 1)
    pos = jax.lax.broadcasted_iota(jnp.int32, (bsz, n), 1)

    size = 2
    while size <= n:
        stride = size // 2
        while stride >= 1:
            up_k = pltpu.roll(key, n - stride, 1)
            dn_k = pltpu.roll(key, stride, 1)
            up_i = pltpu.roll(idx, n - stride, 1)
            dn_i = pltpu.roll(idx, stride, 1)
            high = (pos & stride) != 0
            pk = jnp.where(high, dn_k, up_k)
            pi = jnp.where(high, dn_i, up_i)
            ka = jnp.where(high, pk, key)
            kb = jnp.where(high, key, pk)
            ia = jnp.where(high, pi, idx)
            ib = jnp.where(high, idx, pi)
            prec = (ka > kb) | ((ka == kb) & (ia < ib))
            doswap = prec ^ ((pos & size) == 0)
            key = jnp.where(doswap, pk, key)
            idx = jnp.where(doswap, pi, idx)
            stride //= 2
        size *= 2
    idx_ref[...] = idx[:, :_KEEP]


def _topk_idx(score):
    bsz, n = score.shape
    return pl.pallas_call(
        _topk_body,
        in_specs=[pl.BlockSpec((bsz, n), lambda: (0, 0))],
        out_specs=pl.BlockSpec((bsz, _KEEP), lambda: (0, 0)),
        out_shape=jax.ShapeDtypeStruct((bsz, _KEEP), jnp.int32),
    )(score)


def _make_sc_gather(total_rows, d, nidx):
    from jax.experimental.pallas import tpu_sc as plsc

    info = plsc.get_sparse_core_info()
    nc, ns = info.num_cores, info.num_subcores
    nw = nc * ns
    per_w = nidx // nw
    mesh = plsc.VectorSubcoreMesh(core_axis_name="c", subcore_axis_name="s")

    @functools.partial(
        pl.kernel, mesh=mesh,
        out_type=jax.ShapeDtypeStruct((nidx, d), jnp.float32),
        scratch_types=[
            pltpu.VMEM((per_w,), jnp.int32),
            pltpu.VMEM((per_w, d), jnp.float32),
            pltpu.SemaphoreType.DMA,
        ],
    )
    def k(table_hbm, idx_hbm, out_hbm, idx_v, rows_v, sem):
        wid = lax.axis_index("s") * nc + lax.axis_index("c")
        base = wid * per_w
        pltpu.sync_copy(idx_hbm.at[pl.ds(base, per_w)], idx_v)
        pltpu.async_copy(table_hbm.at[idx_v], rows_v, sem).wait()
        pltpu.sync_copy(rows_v, out_hbm.at[pl.ds(base, per_w)])

    return k


def kernel(tokens, ln_w, ln_b, w, b):
    bsz, n, d = tokens.shape
    k = min(_KEEP, n)

    sumt = jnp.sum(tokens, axis=-1)
    mu = sumt * _INV_DIM
    varsum = jnp.sum(jnp.square(tokens - mu[:, :, None]), axis=-1)

    score = _scores(tokens, sumt, varsum, ln_w, ln_b, w, b)
    keep_idx = _topk_idx(score)

    flat_idx = (keep_idx + (jnp.arange(bsz, dtype=jnp.int32) * n)[:, None])
    flat_idx = flat_idx.reshape(bsz * k)
    table = tokens.reshape(bsz * n, d)
    kept = _make_sc_gather(bsz * n, d, bsz * k)(table, flat_idx)
    kept_tokens = kept.reshape(bsz, k, d)
    kept_mask = jnp.zeros((bsz, k), dtype=jnp.bool_)
    return (kept_tokens, kept_mask, score, keep_idx)


# speedup vs baseline: 2.3132x; 1.0233x over previous
import functools

import jax
import jax.numpy as jnp
import numpy as np
from jax import lax
from jax.experimental import pallas as pl
from jax.experimental.pallas import tpu as pltpu

_DIM = 768
_KEEP = 1024
_ROWS_PER_BLK = 2048
_INV_DIM = np.float32(1.0 / 768.0)
_EPS = np.float32(1e-5)


def _score_body(x_ref, mu_ref, vs_ref, ln_w_ref, ln_b_ref, wpad_ref, b_ref,
                score_ref):
    x = x_ref[0]
    mu = mu_ref[0, 0][:, None]
    sigma = jnp.sqrt(vs_ref[0, 0] * _INV_DIM + _EPS)[:, None]
    h = (x - mu) / sigma * ln_w_ref[...] + ln_b_ref[...]
    s_full = jnp.dot(h, wpad_ref[...], preferred_element_type=jnp.float32)
    score_ref[0, 0] = s_full[:, 0] + b_ref[0]


def _scores(tokens, sumt, varsum, ln_w, ln_b, w, b):
    bsz, n, d = tokens.shape
    rows = bsz * n
    nblk = rows // _ROWS_PER_BLK
    x3 = tokens.reshape(nblk, _ROWS_PER_BLK, d)
    mu3 = (sumt * _INV_DIM).reshape(nblk, 1, _ROWS_PER_BLK)
    vs3 = varsum.reshape(nblk, 1, _ROWS_PER_BLK)
    wpad = jnp.zeros((d, 128), jnp.float32).at[:, 0].set(w[0])
    out = pl.pallas_call(
        _score_body,
        grid=(nblk,),
        in_specs=[
            pl.BlockSpec((1, _ROWS_PER_BLK, d), lambda i: (i, 0, 0)),
            pl.BlockSpec((1, 1, _ROWS_PER_BLK), lambda i: (i, 0, 0)),
            pl.BlockSpec((1, 1, _ROWS_PER_BLK), lambda i: (i, 0, 0)),
            pl.BlockSpec((d,), lambda i: (0,)),
            pl.BlockSpec((d,), lambda i: (0,)),
            pl.BlockSpec((d, 128), lambda i: (0, 0)),
            pl.BlockSpec((1,), lambda i: (0,)),
        ],
        out_specs=pl.BlockSpec((1, 1, _ROWS_PER_BLK), lambda i: (i, 0, 0)),
        out_shape=jax.ShapeDtypeStruct((nblk, 1, _ROWS_PER_BLK), jnp.float32),
    )(x3, mu3, vs3, ln_w, ln_b, wpad, b)
    return out.reshape(bsz, n)


def _topk_body(score_ref, idx_ref):
    s = score_ref[...]
    bsz, n = s.shape
    bits = jax.lax.bitcast_convert_type(s, jnp.int32)
    key = jnp.where(bits < 0, jnp.int32(0x7FFFFFFF) ^ bits, bits)
    idx = jax.lax.broadcasted_iota(jnp.int32, (bsz, n), 1)
    pos = jax.lax.broadcasted_iota(jnp.int32, (bsz, n), 1)

    def step(key, idx, pos, ln, stride, descmask):
        up_k = pltpu.roll(key, ln - stride, 1)
        dn_k = pltpu.roll(key, stride, 1)
        up_i = pltpu.roll(idx, ln - stride, 1)
        dn_i = pltpu.roll(idx, stride, 1)
        high = (pos & stride) != 0
        pk = jnp.where(high, dn_k, up_k)
        pi = jnp.where(high, dn_i, up_i)
        ka = jnp.where(high, pk, key)
        kb = jnp.where(high, key, pk)
        ia = jnp.where(high, pi, idx)
        ib = jnp.where(high, idx, pi)
        prec = (ka > kb) | ((ka == kb) & (ia < ib))
        doswap = prec ^ descmask
        return jnp.where(doswap, pk, key), jnp.where(doswap, pi, idx)

    size = 2
    while size <= _KEEP:
        stride = size // 2
        while stride >= 1:
            key, idx = step(key, idx, pos, n, stride, (pos & size) == 0)
            stride //= 2
        size *= 2

    ln = n
    while ln > _KEEP:
        key, idx = step(key, idx, pos, ln, _KEEP, (pos >= 0))
        key = key.reshape(bsz, ln // (2 * _KEEP), 2, _KEEP)[:, :, 0]
        idx = idx.reshape(bsz, ln // (2 * _KEEP), 2, _KEEP)[:, :, 0]
        ln //= 2
        key = key.reshape(bsz, ln)
        idx = idx.reshape(bsz, ln)
        pos = jax.lax.broadcasted_iota(jnp.int32, (bsz, ln), 1)
        stride = _KEEP // 2
        while stride >= 1:
            key, idx = step(key, idx, pos, ln, stride, (pos & _KEEP) == 0)
            stride //= 2
    idx_ref[...] = idx


def _topk_idx(score):
    bsz, n = score.shape
    return pl.pallas_call(
        _topk_body,
        in_specs=[pl.BlockSpec((bsz, n), lambda: (0, 0))],
        out_specs=pl.BlockSpec((bsz, _KEEP), lambda: (0, 0)),
        out_shape=jax.ShapeDtypeStruct((bsz, _KEEP), jnp.int32),
    )(score)


def _make_sc_gather(total_rows, d, nidx):
    from jax.experimental.pallas import tpu_sc as plsc

    info = plsc.get_sparse_core_info()
    nc, ns = info.num_cores, info.num_subcores
    nw = nc * ns
    per_w = nidx // nw
    mesh = plsc.VectorSubcoreMesh(core_axis_name="c", subcore_axis_name="s")

    @functools.partial(
        pl.kernel, mesh=mesh,
        out_type=jax.ShapeDtypeStruct((nidx, d), jnp.float32),
        scratch_types=[
            pltpu.VMEM((per_w,), jnp.int32),
            pltpu.VMEM((per_w, d), jnp.float32),
            pltpu.SemaphoreType.DMA,
        ],
    )
    def k(table_hbm, idx_hbm, out_hbm, idx_v, rows_v, sem):
        wid = lax.axis_index("s") * nc + lax.axis_index("c")
        base = wid * per_w
        pltpu.sync_copy(idx_hbm.at[pl.ds(base, per_w)], idx_v)
        pltpu.async_copy(table_hbm.at[idx_v], rows_v, sem).wait()
        pltpu.sync_copy(rows_v, out_hbm.at[pl.ds(base, per_w)])

    return k


def kernel(tokens, ln_w, ln_b, w, b):
    bsz, n, d = tokens.shape
    k = min(_KEEP, n)

    sumt = jnp.sum(tokens, axis=-1)
    mu = sumt * _INV_DIM
    varsum = jnp.sum(jnp.square(tokens - mu[:, :, None]), axis=-1)

    score = _scores(tokens, sumt, varsum, ln_w, ln_b, w, b)
    keep_idx = _topk_idx(score)

    flat_idx = (keep_idx + (jnp.arange(bsz, dtype=jnp.int32) * n)[:, None])
    flat_idx = flat_idx.reshape(bsz * k)
    table = tokens.reshape(bsz * n, d)
    kept = _make_sc_gather(bsz * n, d, bsz * k)(table, flat_idx)
    kept_tokens = kept.reshape(bsz, k, d)
    kept_mask = jnp.zeros((bsz, k), dtype=jnp.bool_)
    return (kept_tokens, kept_mask, score, keep_idx)
